# Initial kernel scaffold; baseline (speedup 1.0000x reference)
import jax
import jax.numpy as jnp
from jax.experimental import pallas as pl


def kernel(x, edge_index, edge_weight, W, a):
    raise NotImplementedError("write your pallas kernel here")


# speedup vs baseline: 1.2086x; 1.2086x over previous
import functools

import jax
import jax.numpy as jnp
from jax import lax
from jax.experimental import pallas as pl
from jax.experimental.pallas import tpu as pltpu
from jax.experimental.pallas import tpu_sc as plsc

K = 16
LAMBDA_DECAY = 0.1
LANES = 16


def _tc_body(x_ref, w_ref, dst_ref, ew_ref, h_ref, cn_ref, cw_ref):
    xb = x_ref[...]
    d_in = xb.shape[1]
    decay = jnp.exp(-LAMBDA_DECAY * lax.broadcasted_iota(
        jnp.int32, (1, d_in), 1).astype(jnp.float32))
    h_ref[...] = jnp.dot(xb * decay, w_ref[...], preferred_element_type=jnp.float32)

    w = ew_ref[...]
    dst = dst_ref[...]
    deg = w.shape[1]
    wd = w[:, :, None]
    we = w[:, None, :]
    e_idx = lax.broadcasted_iota(jnp.int32, (1, 1, deg), 2)
    d_idx = lax.broadcasted_iota(jnp.int32, (1, deg, 1), 1)
    beats = (we > wd) | ((we == wd) & (e_idx < d_idx))
    rank = jnp.sum(beats.astype(jnp.int32), axis=2)
    j_idx = lax.broadcasted_iota(jnp.int32, (1, 1, K), 2)
    oh = rank[:, :, None] == j_idx
    cn_ref[...] = jnp.sum(jnp.where(oh, dst[:, :, None], 0), axis=1)
    cw = jnp.sum(jnp.where(oh, w[:, :, None], 0.0), axis=1)
    col = lax.broadcasted_iota(jnp.int32, (K, K * LANES), 1)
    row = lax.broadcasted_iota(jnp.int32, (K, K * LANES), 0)
    rep = (col // LANES == row).astype(jnp.float32)
    cw_ref[...] = jnp.dot(cw, rep, preferred_element_type=jnp.float32)


def _tc_call(xp, W, dstp, ewp):
    np_, d_in = xp.shape
    d_out = W.shape[1]
    deg = dstp.shape[1]
    bn = 256
    grid = np_ // bn
    return pl.pallas_call(
        _tc_body,
        grid=(grid,),
        in_specs=[
            pl.BlockSpec((bn, d_in), lambda i: (i, 0)),
            pl.BlockSpec((d_in, d_out), lambda i: (0, 0)),
            pl.BlockSpec((bn, deg), lambda i: (i, 0)),
            pl.BlockSpec((bn, deg), lambda i: (i, 0)),
        ],
        out_specs=[
            pl.BlockSpec((bn, d_out), lambda i: (i, 0)),
            pl.BlockSpec((bn, K), lambda i: (i, 0)),
            pl.BlockSpec((bn, K * LANES), lambda i: (i, 0)),
        ],
        out_shape=[
            jax.ShapeDtypeStruct((np_, d_out), jnp.float32),
            jax.ShapeDtypeStruct((np_, K), jnp.int32),
            jax.ShapeDtypeStruct((np_, K * LANES), jnp.float32),
        ],
    )(xp, W, dstp, ewp)


def _sc_call(h, idx_flat, w_rep):
    np_, d_out = h.shape
    info = plsc.get_sparse_core_info()
    nc, ns = info.num_cores, info.num_subcores
    nw = nc * ns
    pt = np_ // nw
    c = 8
    r = c * K
    t = pt // c
    nvec = d_out // LANES
    mesh = plsc.VectorSubcoreMesh(core_axis_name="c", subcore_axis_name="s")

    @functools.partial(
        pl.kernel,
        mesh=mesh,
        out_type=jax.ShapeDtypeStruct((np_, d_out), jnp.float32),
        scratch_types=[
            pltpu.VMEM((r,), jnp.int32),
            pltpu.VMEM((r, d_out), jnp.float32),
            pltpu.VMEM((c, K * LANES), jnp.float32),
            pltpu.VMEM((c, d_out), jnp.float32),
            pltpu.SemaphoreType.DMA,
        ],
    )
    def sc_k(h_hbm, idx_hbm, w_hbm, out_hbm, idx_v, rows_v, w_v, out_v, sem):
        wid = lax.axis_index("s") * nc + lax.axis_index("c")

        def chunk_body(tt, carry):
            node0 = wid * pt + tt * c
            base = node0 * K
            pltpu.sync_copy(idx_hbm.at[pl.ds(base, r)], idx_v)
            gat = pltpu.async_copy(h_hbm.at[idx_v], rows_v, sem)
            pltpu.sync_copy(w_hbm.at[pl.ds(node0, c)], w_v)
            gat.wait()

            def node_body(n, carry2):
                def k_body(kk, acc):
                    row = n * K + kk
                    wsplat = w_v[n, pl.ds(kk * LANES, LANES)]
                    return tuple(
                        acc[cc] + wsplat * rows_v[row, pl.ds(cc * LANES, LANES)]
                        for cc in range(nvec))

                zero = jnp.zeros((LANES,), jnp.float32)
                acc = lax.fori_loop(0, K, k_body, (zero,) * nvec)
                for cc in range(nvec):
                    out_v[n, pl.ds(cc * LANES, LANES)] = acc[cc]
                return carry2

            lax.fori_loop(0, c, node_body, 0)
            pltpu.sync_copy(out_v, out_hbm.at[pl.ds(node0, c)])
            return carry

        lax.fori_loop(0, t, chunk_body, 0)

    return sc_k(h, idx_flat, w_rep)


def kernel(x, edge_index, edge_weight, W, a):
    n, d_in = x.shape
    e = edge_index.shape[1]
    deg = e // n
    nw = 32
    c = 8
    np_ = ((n + nw * c - 1) // (nw * c)) * (nw * c)

    dst = edge_index[1].reshape(n, deg)
    ew = edge_weight.reshape(n, deg)
    pad = np_ - n
    xp = jnp.pad(x, ((0, pad), (0, 0)))
    dstp = jnp.pad(dst, ((0, pad), (0, 0)))
    ewp = jnp.pad(ew, ((0, pad), (0, 0)))

    h, cn, cwrep = _tc_call(xp, W, dstp, ewp)
    out = _sc_call(h, cn.reshape(-1), cwrep)
    return out[:n]


# speedup vs baseline: 2.0609x; 1.7052x over previous
import functools

import jax
import jax.numpy as jnp
from jax import lax
from jax.experimental import pallas as pl
from jax.experimental.pallas import tpu as pltpu
from jax.experimental.pallas import tpu_sc as plsc

K = 16
LAMBDA_DECAY = 0.1
LANES = 16


def _tc_body(x_ref, w_ref, dst_ref, ew_ref, h_ref, cn_ref, cw_ref):
    xb = x_ref[...]
    d_in = xb.shape[1]
    decay = jnp.exp(-LAMBDA_DECAY * lax.broadcasted_iota(
        jnp.int32, (1, d_in), 1).astype(jnp.float32))
    h_ref[...] = jnp.dot(xb * decay, w_ref[...], preferred_element_type=jnp.float32)

    w = ew_ref[...]
    dst = dst_ref[...]
    deg = w.shape[1]
    wd = w[:, :, None]
    we = w[:, None, :]
    e_idx = lax.broadcasted_iota(jnp.int32, (1, 1, deg), 2)
    d_idx = lax.broadcasted_iota(jnp.int32, (1, deg, 1), 1)
    beats = (we > wd) | ((we == wd) & (e_idx < d_idx))
    rank = jnp.sum(beats.astype(jnp.int32), axis=2)
    j_idx = lax.broadcasted_iota(jnp.int32, (1, 1, K), 2)
    oh = rank[:, :, None] == j_idx
    cn_ref[...] = jnp.sum(jnp.where(oh, dst[:, :, None], 0), axis=1)
    cw = jnp.sum(jnp.where(oh, w[:, :, None], 0.0), axis=1)
    col = lax.broadcasted_iota(jnp.int32, (K, K * LANES), 1)
    row = lax.broadcasted_iota(jnp.int32, (K, K * LANES), 0)
    rep = (col // LANES == row).astype(jnp.float32)
    cw_ref[...] = jnp.dot(cw, rep, preferred_element_type=jnp.float32)


def _tc_call(xp, W, dstp, ewp):
    np_, d_in = xp.shape
    d_out = W.shape[1]
    deg = dstp.shape[1]
    bn = 256
    grid = np_ // bn
    return pl.pallas_call(
        _tc_body,
        grid=(grid,),
        in_specs=[
            pl.BlockSpec((bn, d_in), lambda i: (i, 0)),
            pl.BlockSpec((d_in, d_out), lambda i: (0, 0)),
            pl.BlockSpec((bn, deg), lambda i: (i, 0)),
            pl.BlockSpec((bn, deg), lambda i: (i, 0)),
        ],
        out_specs=[
            pl.BlockSpec((bn, d_out), lambda i: (i, 0)),
            pl.BlockSpec((bn, K), lambda i: (i, 0)),
            pl.BlockSpec((bn, K * LANES), lambda i: (i, 0)),
        ],
        out_shape=[
            jax.ShapeDtypeStruct((np_, d_out), jnp.float32),
            jax.ShapeDtypeStruct((np_, K), jnp.int32),
            jax.ShapeDtypeStruct((np_, K * LANES), jnp.float32),
        ],
    )(xp, W, dstp, ewp)


def _sc_call(h, idx_flat, w_rep):
    np_, d_out = h.shape
    info = plsc.get_sparse_core_info()
    nc, ns = info.num_cores, info.num_subcores
    nw = nc * ns
    pt = np_ // nw
    c = 16
    r = c * K
    t = pt // c
    nvec = d_out // LANES
    mesh = plsc.VectorSubcoreMesh(core_axis_name="c", subcore_axis_name="s")

    @functools.partial(
        pl.kernel,
        mesh=mesh,
        out_type=jax.ShapeDtypeStruct((np_, d_out), jnp.float32),
        scratch_types=[
            pltpu.VMEM((pt * K,), jnp.int32),
            pltpu.VMEM((2, r, d_out), jnp.float32),
            pltpu.VMEM((2, c, K * LANES), jnp.float32),
            pltpu.VMEM((2, c, d_out), jnp.float32),
            pltpu.SemaphoreType.DMA((2,)),
            pltpu.SemaphoreType.DMA((2,)),
            pltpu.SemaphoreType.DMA((2,)),
        ],
    )
    def sc_k(h_hbm, idx_hbm, w_hbm, out_hbm, idx_v, rows_v, w_v, out_v,
             gsem, wsem, osem):
        wid = lax.axis_index("s") * nc + lax.axis_index("c")
        wnode0 = wid * pt

        def fetch(tt, b):
            pltpu.async_copy(
                h_hbm.at[idx_v.at[pl.ds(tt * r, r)]], rows_v.at[b], gsem.at[b])
            pltpu.async_copy(
                w_hbm.at[pl.ds(wnode0 + tt * c, c)], w_v.at[b], wsem.at[b])

        pltpu.sync_copy(idx_hbm.at[pl.ds(wnode0 * K, pt * K)], idx_v)
        fetch(0, 0)

        def pair_body(t2, carry):
            for b in range(2):
                tt = t2 * 2 + b
                ob = 1 - b

                @pl.when(tt + 1 < t)
                def _():
                    fetch(tt + 1, ob)

                pltpu.make_async_copy(
                    h_hbm.at[idx_v.at[pl.ds(tt * r, r)]], rows_v.at[b],
                    gsem.at[b]).wait()
                pltpu.make_async_copy(
                    w_hbm.at[pl.ds(wnode0 + tt * c, c)], w_v.at[b],
                    wsem.at[b]).wait()

                def node_body(n, carry2):
                    acc = [None] * nvec
                    for kk in range(K):
                        row = n * K + kk
                        wsplat = w_v[b, n, pl.ds(kk * LANES, LANES)]
                        for cc in range(nvec):
                            term = wsplat * rows_v[b, row, pl.ds(cc * LANES, LANES)]
                            acc[cc] = term if kk == 0 else acc[cc] + term
                    for cc in range(nvec):
                        out_v[b, n, pl.ds(cc * LANES, LANES)] = acc[cc]
                    return carry2

                lax.fori_loop(0, c, node_body, 0)

                @pl.when(tt >= 2)
                def _():
                    pltpu.make_async_copy(
                        out_v.at[b],
                        out_hbm.at[pl.ds(wnode0 + (tt - 2) * c, c)],
                        osem.at[b]).wait()

                pltpu.async_copy(
                    out_v.at[b], out_hbm.at[pl.ds(wnode0 + tt * c, c)],
                    osem.at[b])
            return carry

        lax.fori_loop(0, t // 2, pair_body, 0)
        for b in range(2):
            pltpu.make_async_copy(
                out_v.at[b], out_hbm.at[pl.ds(wnode0 + (t - 2 + b) * c, c)],
                osem.at[b]).wait()

    return sc_k(h, idx_flat, w_rep)


def kernel(x, edge_index, edge_weight, W, a):
    n, d_in = x.shape
    e = edge_index.shape[1]
    deg = e // n
    nw = 32
    c = 8
    np_ = ((n + nw * c - 1) // (nw * c)) * (nw * c)

    dst = edge_index[1].reshape(n, deg)
    ew = edge_weight.reshape(n, deg)
    pad = np_ - n
    xp = jnp.pad(x, ((0, pad), (0, 0)))
    pad_dst = (jnp.arange(pad * deg, dtype=jnp.int32) % n).reshape(pad, deg)
    dstp = jnp.concatenate([dst, pad_dst], axis=0)
    ewp = jnp.pad(ew, ((0, pad), (0, 0)))

    h, cn, cwrep = _tc_call(xp, W, dstp, ewp)
    out = _sc_call(h, cn.reshape(-1), cwrep)
    return out[:n]


# speedup vs baseline: 4.3828x; 2.1266x over previous
import functools

import jax
import jax.numpy as jnp
from jax import lax
from jax.experimental import pallas as pl
from jax.experimental.pallas import tpu as pltpu
from jax.experimental.pallas import tpu_sc as plsc

K = 16
LAMBDA_DECAY = 0.1
LANES = 16


def _tc_body(x_ref, w_ref, dstf_ref, ew_ref, h_ref, cn_ref, cw_ref):
    xb = x_ref[...]
    d_in = xb.shape[1]
    decay = jnp.exp(-LAMBDA_DECAY * lax.broadcasted_iota(
        jnp.int32, (1, d_in), 1).astype(jnp.float32))
    h_ref[...] = jnp.dot(xb * decay, w_ref[...], preferred_element_type=jnp.float32)

    wt = ew_ref[...]
    dft = dstf_ref[...]
    deg = wt.shape[0]
    we = wt[:, None, :]
    wd = wt[None, :, :]
    e_i = lax.broadcasted_iota(jnp.int32, (deg, 1, 1), 0)
    d_i = lax.broadcasted_iota(jnp.int32, (1, deg, 1), 1)
    tie = e_i < d_i
    beats = jnp.where((we > wd) | ((we == wd) & tie), 1.0, 0.0)
    rank = jnp.sum(beats, axis=0)
    j_i = lax.broadcasted_iota(jnp.int32, (1, K, 1), 1).astype(jnp.float32)
    ohf = jnp.where(rank[:, None, :] == j_i, 1.0, 0.0)
    cnf = jnp.sum(ohf * dft[:, None, :], axis=0)
    cwk = jnp.sum(ohf * wt[:, None, :], axis=0)
    cn_ref[...] = cnf.T.astype(jnp.int32)
    col = lax.broadcasted_iota(jnp.int32, (K, K * LANES), 1)
    row = lax.broadcasted_iota(jnp.int32, (K, K * LANES), 0)
    rep = (col // LANES == row).astype(jnp.float32)
    cw_ref[...] = jnp.dot(cwk.T, rep, preferred_element_type=jnp.float32)


def _tc_call(xp, W, dstp, ewp):
    np_, d_in = xp.shape
    d_out = W.shape[1]
    deg = dstp.shape[0]
    bn = 256
    grid = np_ // bn
    return pl.pallas_call(
        _tc_body,
        grid=(grid,),
        in_specs=[
            pl.BlockSpec((bn, d_in), lambda i: (i, 0)),
            pl.BlockSpec((d_in, d_out), lambda i: (0, 0)),
            pl.BlockSpec((deg, bn), lambda i: (0, i)),
            pl.BlockSpec((deg, bn), lambda i: (0, i)),
        ],
        out_specs=[
            pl.BlockSpec((bn, d_out), lambda i: (i, 0)),
            pl.BlockSpec((bn, K), lambda i: (i, 0)),
            pl.BlockSpec((bn, K * LANES), lambda i: (i, 0)),
        ],
        out_shape=[
            jax.ShapeDtypeStruct((np_, d_out), jnp.float32),
            jax.ShapeDtypeStruct((np_, K), jnp.int32),
            jax.ShapeDtypeStruct((np_, K * LANES), jnp.float32),
        ],
    )(xp, W, dstp, ewp)


def _sc_call(h, idx_flat, w_rep):
    np_, d_out = h.shape
    info = plsc.get_sparse_core_info()
    nc, ns = info.num_cores, info.num_subcores
    nw = nc * ns
    pt = np_ // nw
    c = 16
    r = c * K
    t = pt // c
    nvec = d_out // LANES
    mesh = plsc.VectorSubcoreMesh(core_axis_name="c", subcore_axis_name="s")

    @functools.partial(
        pl.kernel,
        mesh=mesh,
        out_type=jax.ShapeDtypeStruct((np_, d_out), jnp.float32),
        scratch_types=[
            pltpu.VMEM((pt * K,), jnp.int32),
            pltpu.VMEM((2, r, d_out), jnp.float32),
            pltpu.VMEM((2, c, K * LANES), jnp.float32),
            pltpu.VMEM((2, c, d_out), jnp.float32),
            pltpu.SemaphoreType.DMA((2,)),
            pltpu.SemaphoreType.DMA((2,)),
            pltpu.SemaphoreType.DMA((2,)),
        ],
    )
    def sc_k(h_hbm, idx_hbm, w_hbm, out_hbm, idx_v, rows_v, w_v, out_v,
             gsem, wsem, osem):
        wid = lax.axis_index("s") * nc + lax.axis_index("c")
        wnode0 = wid * pt

        def fetch(tt, b):
            pltpu.async_copy(
                h_hbm.at[idx_v.at[pl.ds(tt * r, r)]], rows_v.at[b], gsem.at[b])
            pltpu.async_copy(
                w_hbm.at[pl.ds(wnode0 + tt * c, c)], w_v.at[b], wsem.at[b])

        pltpu.sync_copy(idx_hbm.at[pl.ds(wnode0 * K, pt * K)], idx_v)
        fetch(0, 0)

        def pair_body(t2, carry):
            for b in range(2):
                tt = t2 * 2 + b
                ob = 1 - b

                @pl.when(tt + 1 < t)
                def _():
                    fetch(tt + 1, ob)

                pltpu.make_async_copy(
                    h_hbm.at[idx_v.at[pl.ds(tt * r, r)]], rows_v.at[b],
                    gsem.at[b]).wait()
                pltpu.make_async_copy(
                    w_hbm.at[pl.ds(wnode0 + tt * c, c)], w_v.at[b],
                    wsem.at[b]).wait()

                def node_body(n, carry2):
                    acc = [None] * nvec
                    for kk in range(K):
                        row = n * K + kk
                        wsplat = w_v[b, n, pl.ds(kk * LANES, LANES)]
                        for cc in range(nvec):
                            term = wsplat * rows_v[b, row, pl.ds(cc * LANES, LANES)]
                            acc[cc] = term if kk == 0 else acc[cc] + term
                    for cc in range(nvec):
                        out_v[b, n, pl.ds(cc * LANES, LANES)] = acc[cc]
                    return carry2

                lax.fori_loop(0, c, node_body, 0)

                @pl.when(tt >= 2)
                def _():
                    pltpu.make_async_copy(
                        out_v.at[b],
                        out_hbm.at[pl.ds(wnode0 + (tt - 2) * c, c)],
                        osem.at[b]).wait()

                pltpu.async_copy(
                    out_v.at[b], out_hbm.at[pl.ds(wnode0 + tt * c, c)],
                    osem.at[b])
            return carry

        lax.fori_loop(0, t // 2, pair_body, 0)
        for b in range(2):
            pltpu.make_async_copy(
                out_v.at[b], out_hbm.at[pl.ds(wnode0 + (t - 2 + b) * c, c)],
                osem.at[b]).wait()

    return sc_k(h, idx_flat, w_rep)


def kernel(x, edge_index, edge_weight, W, a):
    n, d_in = x.shape
    e = edge_index.shape[1]
    deg = e // n
    nw = 32
    c = 8
    np_ = ((n + nw * c - 1) // (nw * c)) * (nw * c)

    dst = edge_index[1].reshape(n, deg)
    ew = edge_weight.reshape(n, deg)
    pad = np_ - n
    xp = jnp.pad(x, ((0, pad), (0, 0)))
    pad_dst = (jnp.arange(pad * deg, dtype=jnp.int32) % n).reshape(pad, deg)
    dstp = jnp.concatenate([dst, pad_dst], axis=0).astype(jnp.float32).T
    ewp = jnp.pad(ew, ((0, pad), (0, 0))).T

    h, cn, cwrep = _tc_call(xp, W, dstp, ewp)
    out = _sc_call(h, cn.reshape(-1), cwrep)
    return out[:n]


# speedup vs baseline: 4.5216x; 1.0317x over previous
import functools

import jax
import jax.numpy as jnp
from jax import lax
from jax.experimental import pallas as pl
from jax.experimental.pallas import tpu as pltpu
from jax.experimental.pallas import tpu_sc as plsc

K = 16
LAMBDA_DECAY = 0.1
LANES = 16


def _tc_body(x_ref, w_ref, dstf_ref, ew_ref, h_ref, cn_ref, cw_ref):
    xb = x_ref[...]
    d_in = xb.shape[1]
    decay = jnp.exp(-LAMBDA_DECAY * lax.broadcasted_iota(
        jnp.int32, (1, d_in), 1).astype(jnp.float32))
    h_ref[...] = jnp.dot(xb * decay, w_ref[...], preferred_element_type=jnp.float32)

    wt = ew_ref[...]
    dft = dstf_ref[...]
    deg = wt.shape[0]
    we = wt[:, None, :]
    wd = wt[None, :, :]
    e_i = lax.broadcasted_iota(jnp.int32, (deg, 1, 1), 0)
    d_i = lax.broadcasted_iota(jnp.int32, (1, deg, 1), 1)
    tie = e_i < d_i
    beats = jnp.where((we > wd) | ((we == wd) & tie), 1.0, 0.0)
    rank = jnp.sum(beats, axis=0)
    j_i = lax.broadcasted_iota(jnp.int32, (1, K, 1), 1).astype(jnp.float32)
    ohf = jnp.where(rank[:, None, :] == j_i, 1.0, 0.0)
    cnf = jnp.sum(ohf * dft[:, None, :], axis=0)
    cwk = jnp.sum(ohf * wt[:, None, :], axis=0)
    cn_ref[...] = cnf.T.astype(jnp.int32)
    cw_ref[...] = cwk.T


def _tc_call(xp, W, dstp, ewp):
    np_, d_in = xp.shape
    d_out = W.shape[1]
    deg = dstp.shape[0]
    bn = 256
    grid = np_ // bn
    return pl.pallas_call(
        _tc_body,
        grid=(grid,),
        in_specs=[
            pl.BlockSpec((bn, d_in), lambda i: (i, 0)),
            pl.BlockSpec((d_in, d_out), lambda i: (0, 0)),
            pl.BlockSpec((deg, bn), lambda i: (0, i)),
            pl.BlockSpec((deg, bn), lambda i: (0, i)),
        ],
        out_specs=[
            pl.BlockSpec((bn, d_out), lambda i: (i, 0)),
            pl.BlockSpec((bn, K), lambda i: (i, 0)),
            pl.BlockSpec((bn, K), lambda i: (i, 0)),
        ],
        out_shape=[
            jax.ShapeDtypeStruct((np_, d_out), jnp.float32),
            jax.ShapeDtypeStruct((np_, K), jnp.int32),
            jax.ShapeDtypeStruct((np_, K), jnp.float32),
        ],
    )(xp, W, dstp, ewp)


def _splat(vec, k):
    idx = jnp.full((LANES, 1), k, dtype=jnp.int32)
    dn = lax.GatherDimensionNumbers(
        offset_dims=(), collapsed_slice_dims=(0,), start_index_map=(0,))
    return lax.gather(vec, idx, dn, slice_sizes=(1,),
                      mode=lax.GatherScatterMode.PROMISE_IN_BOUNDS)


def _sc_call(h, idx_flat, w_rep):
    np_, d_out = h.shape
    info = plsc.get_sparse_core_info()
    nc, ns = info.num_cores, info.num_subcores
    nw = nc * ns
    pt = np_ // nw
    c = 16
    r = c * K
    t = pt // c
    nvec = d_out // LANES
    mesh = plsc.VectorSubcoreMesh(core_axis_name="c", subcore_axis_name="s")

    @functools.partial(
        pl.kernel,
        mesh=mesh,
        out_type=jax.ShapeDtypeStruct((np_, d_out), jnp.float32),
        scratch_types=[
            pltpu.VMEM((pt * K,), jnp.int32),
            pltpu.VMEM((2, r, d_out), jnp.float32),
            pltpu.VMEM((2, c, K), jnp.float32),
            pltpu.VMEM((2, c, d_out), jnp.float32),
            pltpu.SemaphoreType.DMA((2,)),
            pltpu.SemaphoreType.DMA((2,)),
            pltpu.SemaphoreType.DMA((2,)),
        ],
    )
    def sc_k(h_hbm, idx_hbm, w_hbm, out_hbm, idx_v, rows_v, w_v, out_v,
             gsem, wsem, osem):
        wid = lax.axis_index("s") * nc + lax.axis_index("c")
        wnode0 = wid * pt

        def fetch(tt, b):
            pltpu.async_copy(
                h_hbm.at[idx_v.at[pl.ds(tt * r, r)]], rows_v.at[b], gsem.at[b])
            pltpu.async_copy(
                w_hbm.at[pl.ds(wnode0 + tt * c, c)], w_v.at[b], wsem.at[b])

        pltpu.sync_copy(idx_hbm.at[pl.ds(wnode0 * K, pt * K)], idx_v)
        fetch(0, 0)

        def pair_body(t2, carry):
            for b in range(2):
                tt = t2 * 2 + b
                ob = 1 - b

                @pl.when(tt + 1 < t)
                def _():
                    fetch(tt + 1, ob)

                pltpu.make_async_copy(
                    h_hbm.at[idx_v.at[pl.ds(tt * r, r)]], rows_v.at[b],
                    gsem.at[b]).wait()
                pltpu.make_async_copy(
                    w_hbm.at[pl.ds(wnode0 + tt * c, c)], w_v.at[b],
                    wsem.at[b]).wait()

                def node_body(n, carry2):
                    acc = [None] * nvec
                    wrow = w_v[b, n, :]
                    for kk in range(K):
                        row = n * K + kk
                        wsplat = _splat(wrow, kk)
                        for cc in range(nvec):
                            term = wsplat * rows_v[b, row, pl.ds(cc * LANES, LANES)]
                            acc[cc] = term if kk == 0 else acc[cc] + term
                    for cc in range(nvec):
                        out_v[b, n, pl.ds(cc * LANES, LANES)] = acc[cc]
                    return carry2

                lax.fori_loop(0, c, node_body, 0)

                @pl.when(tt >= 2)
                def _():
                    pltpu.make_async_copy(
                        out_v.at[b],
                        out_hbm.at[pl.ds(wnode0 + (tt - 2) * c, c)],
                        osem.at[b]).wait()

                pltpu.async_copy(
                    out_v.at[b], out_hbm.at[pl.ds(wnode0 + tt * c, c)],
                    osem.at[b])
            return carry

        lax.fori_loop(0, t // 2, pair_body, 0)
        for b in range(2):
            pltpu.make_async_copy(
                out_v.at[b], out_hbm.at[pl.ds(wnode0 + (t - 2 + b) * c, c)],
                osem.at[b]).wait()

    return sc_k(h, idx_flat, w_rep)


def kernel(x, edge_index, edge_weight, W, a):
    n, d_in = x.shape
    e = edge_index.shape[1]
    deg = e // n
    nw = 32
    c = 8
    np_ = ((n + nw * c - 1) // (nw * c)) * (nw * c)

    dst = edge_index[1].reshape(n, deg)
    ew = edge_weight.reshape(n, deg)
    pad = np_ - n
    xp = jnp.pad(x, ((0, pad), (0, 0)))
    pad_dst = (jnp.arange(pad * deg, dtype=jnp.int32) % n).reshape(pad, deg)
    dstp = jnp.concatenate([dst, pad_dst], axis=0).astype(jnp.float32).T
    ewp = jnp.pad(ew, ((0, pad), (0, 0))).T

    h, cn, cwrep = _tc_call(xp, W, dstp, ewp)
    out = _sc_call(h, cn.reshape(-1), cwrep)
    return out[:n]


# speedup vs baseline: 4.6297x; 1.0239x over previous
import functools

import jax
import jax.numpy as jnp
from jax import lax
from jax.experimental import pallas as pl
from jax.experimental.pallas import tpu as pltpu
from jax.experimental.pallas import tpu_sc as plsc

K = 16
LAMBDA_DECAY = 0.1
LANES = 16


def _tc_body(x_ref, w_ref, dstf_ref, ew_ref, h_ref, cn_ref, cw_ref):
    xb = x_ref[...]
    d_in = xb.shape[1]
    decay = jnp.exp(-LAMBDA_DECAY * lax.broadcasted_iota(
        jnp.int32, (1, d_in), 1).astype(jnp.float32))
    h_ref[...] = jnp.dot(xb * decay, w_ref[...], preferred_element_type=jnp.float32)

    wt = ew_ref[...].T
    dft = dstf_ref[...].astype(jnp.float32).T
    deg = wt.shape[0]
    we = wt[:, None, :]
    wd = wt[None, :, :]
    e_i = lax.broadcasted_iota(jnp.int32, (deg, 1, 1), 0)
    d_i = lax.broadcasted_iota(jnp.int32, (1, deg, 1), 1)
    tie = e_i < d_i
    beats = jnp.where((we > wd) | ((we == wd) & tie), 1.0, 0.0)
    rank = jnp.sum(beats, axis=0)
    j_i = lax.broadcasted_iota(jnp.int32, (1, K, 1), 1).astype(jnp.float32)
    ohf = jnp.where(rank[:, None, :] == j_i, 1.0, 0.0)
    cnf = jnp.sum(ohf * dft[:, None, :], axis=0)
    cwk = jnp.sum(ohf * wt[:, None, :], axis=0)
    cn_ref[...] = cnf.T.astype(jnp.int32)
    cw_ref[...] = cwk.T


def _tc_call(x, W, dstp, ewp):
    n, d_in = x.shape
    d_out = W.shape[1]
    np_, deg = dstp.shape
    bn = 256
    grid = np_ // bn
    return pl.pallas_call(
        _tc_body,
        grid=(grid,),
        in_specs=[
            pl.BlockSpec((bn, d_in), lambda i: (i, 0)),
            pl.BlockSpec((d_in, d_out), lambda i: (0, 0)),
            pl.BlockSpec((bn, deg), lambda i: (i, 0)),
            pl.BlockSpec((bn, deg), lambda i: (i, 0)),
        ],
        out_specs=[
            pl.BlockSpec((bn, d_out), lambda i: (i, 0)),
            pl.BlockSpec((bn, K), lambda i: (i, 0)),
            pl.BlockSpec((bn, K), lambda i: (i, 0)),
        ],
        out_shape=[
            jax.ShapeDtypeStruct((n, d_out), jnp.float32),
            jax.ShapeDtypeStruct((np_, K), jnp.int32),
            jax.ShapeDtypeStruct((np_, K), jnp.float32),
        ],
    )(x, W, dstp, ewp)


def _splat(vec, k):
    idx = jnp.full((LANES, 1), k, dtype=jnp.int32)
    dn = lax.GatherDimensionNumbers(
        offset_dims=(), collapsed_slice_dims=(0,), start_index_map=(0,))
    return lax.gather(vec, idx, dn, slice_sizes=(1,),
                      mode=lax.GatherScatterMode.PROMISE_IN_BOUNDS)


def _sc_call(h, idx_flat, w_rep):
    d_out = h.shape[1]
    np_ = w_rep.shape[0]
    info = plsc.get_sparse_core_info()
    nc, ns = info.num_cores, info.num_subcores
    nw = nc * ns
    pt = np_ // nw
    c = 16
    r = c * K
    t = pt // c
    nvec = d_out // LANES
    mesh = plsc.VectorSubcoreMesh(core_axis_name="c", subcore_axis_name="s")

    @functools.partial(
        pl.kernel,
        mesh=mesh,
        out_type=jax.ShapeDtypeStruct((np_, d_out), jnp.float32),
        scratch_types=[
            pltpu.VMEM((pt * K,), jnp.int32),
            pltpu.VMEM((2, r, d_out), jnp.float32),
            pltpu.VMEM((2, c, K), jnp.float32),
            pltpu.VMEM((2, c, d_out), jnp.float32),
            pltpu.SemaphoreType.DMA((2,)),
            pltpu.SemaphoreType.DMA((2,)),
            pltpu.SemaphoreType.DMA((2,)),
        ],
    )
    def sc_k(h_hbm, idx_hbm, w_hbm, out_hbm, idx_v, rows_v, w_v, out_v,
             gsem, wsem, osem):
        wid = lax.axis_index("s") * nc + lax.axis_index("c")
        wnode0 = wid * pt

        def fetch(tt, b):
            pltpu.async_copy(
                h_hbm.at[idx_v.at[pl.ds(tt * r, r)]], rows_v.at[b], gsem.at[b])
            pltpu.async_copy(
                w_hbm.at[pl.ds(wnode0 + tt * c, c)], w_v.at[b], wsem.at[b])

        pltpu.sync_copy(idx_hbm.at[pl.ds(wnode0 * K, pt * K)], idx_v)
        fetch(0, 0)

        def pair_body(t2, carry):
            for b in range(2):
                tt = t2 * 2 + b
                ob = 1 - b

                @pl.when(tt + 1 < t)
                def _():
                    fetch(tt + 1, ob)

                pltpu.make_async_copy(
                    h_hbm.at[idx_v.at[pl.ds(tt * r, r)]], rows_v.at[b],
                    gsem.at[b]).wait()
                pltpu.make_async_copy(
                    w_hbm.at[pl.ds(wnode0 + tt * c, c)], w_v.at[b],
                    wsem.at[b]).wait()

                def node_body(n, carry2):
                    acc = [None] * nvec
                    wrow = w_v[b, n, :]
                    for kk in range(K):
                        row = n * K + kk
                        wsplat = _splat(wrow, kk)
                        for cc in range(nvec):
                            term = wsplat * rows_v[b, row, pl.ds(cc * LANES, LANES)]
                            acc[cc] = term if kk == 0 else acc[cc] + term
                    for cc in range(nvec):
                        out_v[b, n, pl.ds(cc * LANES, LANES)] = acc[cc]
                    return carry2

                lax.fori_loop(0, c, node_body, 0)

                @pl.when(tt >= 2)
                def _():
                    pltpu.make_async_copy(
                        out_v.at[b],
                        out_hbm.at[pl.ds(wnode0 + (tt - 2) * c, c)],
                        osem.at[b]).wait()

                pltpu.async_copy(
                    out_v.at[b], out_hbm.at[pl.ds(wnode0 + tt * c, c)],
                    osem.at[b])
            return carry

        lax.fori_loop(0, t // 2, pair_body, 0)
        for b in range(2):
            pltpu.make_async_copy(
                out_v.at[b], out_hbm.at[pl.ds(wnode0 + (t - 2 + b) * c, c)],
                osem.at[b]).wait()

    return sc_k(h, idx_flat, w_rep)


def kernel(x, edge_index, edge_weight, W, a):
    n, d_in = x.shape
    e = edge_index.shape[1]
    deg = e // n
    nw = 32
    c = 8
    np_ = ((n + nw * c - 1) // (nw * c)) * (nw * c)

    dst = edge_index[1].reshape(n, deg)
    ew = edge_weight.reshape(n, deg)
    pad = np_ - n
    pad_dst = (jnp.arange(pad * deg, dtype=jnp.int32) % n).reshape(pad, deg)
    dstp = jnp.concatenate([dst, pad_dst], axis=0)
    ewp = jnp.pad(ew, ((0, pad), (0, 0)))

    h, cn, cw = _tc_call(x, W, dstp, ewp)
    out = _sc_call(h, cn.reshape(-1), cw)
    return out[:n]


# speedup vs baseline: 4.8600x; 1.0497x over previous
import functools

import jax
import jax.numpy as jnp
from jax import lax
from jax.experimental import pallas as pl
from jax.experimental.pallas import tpu as pltpu
from jax.experimental.pallas import tpu_sc as plsc

K = 16
LAMBDA_DECAY = 0.1
LANES = 16


def _tc_body(n_nodes, deg, x_ref, w_ref, ei_ref, ew_ref, h_ref, cn_ref, cw_ref):
    xb = x_ref[...]
    d_in = xb.shape[1]
    decay = jnp.exp(-LAMBDA_DECAY * lax.broadcasted_iota(
        jnp.int32, (1, d_in), 1).astype(jnp.float32))
    h_ref[...] = jnp.dot(xb * decay, w_ref[...], preferred_element_type=jnp.float32)

    bn = cn_ref.shape[0]
    wt = ew_ref[...].T
    dft = ei_ref[...].astype(jnp.float32).T
    node = lax.broadcasted_iota(
        jnp.int32, (1, bn), 1) + pl.program_id(0) * bn
    e_i0 = lax.broadcasted_iota(jnp.int32, (deg, 1), 0)
    valid = node < n_nodes
    spread = ((node * deg + e_i0) % n_nodes).astype(jnp.float32)
    wt = jnp.where(valid, wt, 0.0)
    dft = jnp.where(valid, dft, spread)
    we = wt[:, None, :]
    wd = wt[None, :, :]
    e_i = lax.broadcasted_iota(jnp.int32, (deg, 1, 1), 0)
    d_i = lax.broadcasted_iota(jnp.int32, (1, deg, 1), 1)
    tie = e_i < d_i
    beats = jnp.where((we > wd) | ((we == wd) & tie), 1.0, 0.0)
    rank = jnp.sum(beats, axis=0)
    j_i = lax.broadcasted_iota(jnp.int32, (1, K, 1), 1).astype(jnp.float32)
    ohf = jnp.where(rank[:, None, :] == j_i, 1.0, 0.0)
    cnf = jnp.sum(ohf * dft[:, None, :], axis=0)
    cwk = jnp.sum(ohf * wt[:, None, :], axis=0)
    cn_ref[...] = cnf.T.astype(jnp.int32)
    cw_ref[...] = cwk.T


def _tc_call(x, W, dst, edge_weight, np_):
    n, d_in = x.shape
    d_out = W.shape[1]
    deg = dst.shape[1]
    bn = 256
    grid = np_ // bn
    return pl.pallas_call(
        functools.partial(_tc_body, n, deg),
        grid=(grid,),
        in_specs=[
            pl.BlockSpec((bn, d_in), lambda i: (i, 0)),
            pl.BlockSpec((d_in, d_out), lambda i: (0, 0)),
            pl.BlockSpec((bn, deg), lambda i: (i, 0)),
            pl.BlockSpec((bn, deg), lambda i: (i, 0)),
        ],
        out_specs=[
            pl.BlockSpec((bn, d_out), lambda i: (i, 0)),
            pl.BlockSpec((bn, K), lambda i: (i, 0)),
            pl.BlockSpec((bn, K), lambda i: (i, 0)),
        ],
        out_shape=[
            jax.ShapeDtypeStruct((n, d_out), jnp.float32),
            jax.ShapeDtypeStruct((np_, K), jnp.int32),
            jax.ShapeDtypeStruct((np_, K), jnp.float32),
        ],
    )(x, W, dst, edge_weight)


def _splat(vec, k):
    idx = jnp.full((LANES, 1), k, dtype=jnp.int32)
    dn = lax.GatherDimensionNumbers(
        offset_dims=(), collapsed_slice_dims=(0,), start_index_map=(0,))
    return lax.gather(vec, idx, dn, slice_sizes=(1,),
                      mode=lax.GatherScatterMode.PROMISE_IN_BOUNDS)


def _sc_call(h, idx_flat, w_rep):
    d_out = h.shape[1]
    np_ = w_rep.shape[0]
    info = plsc.get_sparse_core_info()
    nc, ns = info.num_cores, info.num_subcores
    nw = nc * ns
    pt = np_ // nw
    c = 16
    r = c * K
    t = pt // c
    nvec = d_out // LANES
    mesh = plsc.VectorSubcoreMesh(core_axis_name="c", subcore_axis_name="s")

    @functools.partial(
        pl.kernel,
        mesh=mesh,
        out_type=jax.ShapeDtypeStruct((np_, d_out), jnp.float32),
        scratch_types=[
            pltpu.VMEM((pt * K,), jnp.int32),
            pltpu.VMEM((2, r, d_out), jnp.float32),
            pltpu.VMEM((pt, K), jnp.float32),
            pltpu.VMEM((2, c, d_out), jnp.float32),
            pltpu.SemaphoreType.DMA((2,)),
            pltpu.SemaphoreType.DMA((2,)),
        ],
    )
    def sc_k(h_hbm, idx_hbm, w_hbm, out_hbm, idx_v, rows_v, w_v, out_v,
             gsem, osem):
        wid = lax.axis_index("s") * nc + lax.axis_index("c")
        wnode0 = wid * pt

        def fetch(tt, b):
            pltpu.async_copy(
                h_hbm.at[idx_v.at[pl.ds(tt * r, r)]], rows_v.at[b], gsem.at[b])

        pltpu.sync_copy(idx_hbm.at[pl.ds(wnode0 * K, pt * K)], idx_v)
        pltpu.sync_copy(w_hbm.at[pl.ds(wnode0, pt)], w_v)
        fetch(0, 0)

        def pair_body(t2, carry):
            for b in range(2):
                tt = t2 * 2 + b
                ob = 1 - b

                @pl.when(tt + 1 < t)
                def _():
                    fetch(tt + 1, ob)

                pltpu.make_async_copy(
                    h_hbm.at[idx_v.at[pl.ds(tt * r, r)]], rows_v.at[b],
                    gsem.at[b]).wait()

                def node_body(n, carry2):
                    acc = [None] * nvec
                    wrow = w_v[tt * c + n, :]
                    for kk in range(K):
                        row = n * K + kk
                        wsplat = _splat(wrow, kk)
                        for cc in range(nvec):
                            term = wsplat * rows_v[b, row, pl.ds(cc * LANES, LANES)]
                            acc[cc] = term if kk == 0 else acc[cc] + term
                    for cc in range(nvec):
                        out_v[b, n, pl.ds(cc * LANES, LANES)] = acc[cc]
                    return carry2

                lax.fori_loop(0, c, node_body, 0)

                @pl.when(tt >= 2)
                def _():
                    pltpu.make_async_copy(
                        out_v.at[b],
                        out_hbm.at[pl.ds(wnode0 + (tt - 2) * c, c)],
                        osem.at[b]).wait()

                pltpu.async_copy(
                    out_v.at[b], out_hbm.at[pl.ds(wnode0 + tt * c, c)],
                    osem.at[b])
            return carry

        lax.fori_loop(0, t // 2, pair_body, 0)
        for b in range(2):
            pltpu.make_async_copy(
                out_v.at[b], out_hbm.at[pl.ds(wnode0 + (t - 2 + b) * c, c)],
                osem.at[b]).wait()

    return sc_k(h, idx_flat, w_rep)


def kernel(x, edge_index, edge_weight, W, a):
    n, d_in = x.shape
    e = edge_index.shape[1]
    deg = e // n
    nw = 32
    c = 8
    np_ = ((n + nw * c - 1) // (nw * c)) * (nw * c)

    h, cn, cw = _tc_call(x, W, edge_index[1].reshape(n, deg),
                         edge_weight.reshape(n, deg), np_)
    out = _sc_call(h, cn.reshape(-1), cw)
    return out[:n]


# speedup vs baseline: 5.3703x; 1.1050x over previous
import functools

import jax
import jax.numpy as jnp
from jax import lax
from jax.experimental import pallas as pl
from jax.experimental.pallas import tpu as pltpu
from jax.experimental.pallas import tpu_sc as plsc

K = 16
LAMBDA_DECAY = 0.1
LANES = 16


def _tc_body(n_nodes, deg, x_ref, w_ref, ei_ref, ew_ref, h_ref, cn_ref, cw_ref):
    xb = x_ref[...]
    d_in = xb.shape[1]
    decay = jnp.exp(-LAMBDA_DECAY * lax.broadcasted_iota(
        jnp.int32, (1, d_in), 1).astype(jnp.float32))
    h_ref[...] = jnp.dot(xb * decay, w_ref[...], preferred_element_type=jnp.float32)

    bn = cn_ref.shape[0]
    wt = ew_ref[...].T
    dft = ei_ref[1].astype(jnp.float32).T
    node = lax.broadcasted_iota(
        jnp.int32, (1, bn), 1) + pl.program_id(0) * bn
    e_i0 = lax.broadcasted_iota(jnp.int32, (deg, 1), 0)
    valid = node < n_nodes
    spread = ((node * deg + e_i0) % n_nodes).astype(jnp.float32)
    wt = jnp.where(valid, wt, 0.0)
    dft = jnp.where(valid, dft, spread)
    we = wt[:, None, :]
    wd = wt[None, :, :]
    e_i = lax.broadcasted_iota(jnp.int32, (deg, 1, 1), 0)
    d_i = lax.broadcasted_iota(jnp.int32, (1, deg, 1), 1)
    tie = e_i < d_i
    beats = jnp.where((we > wd) | ((we == wd) & tie), 1.0, 0.0)
    rank = jnp.sum(beats, axis=0)
    j_i = lax.broadcasted_iota(jnp.int32, (1, K, 1), 1).astype(jnp.float32)
    ohf = jnp.where(rank[:, None, :] == j_i, 1.0, 0.0)
    cnf = jnp.sum(ohf * dft[:, None, :], axis=0)
    cwk = jnp.sum(ohf * wt[:, None, :], axis=0)
    cn_ref[...] = cnf.T.astype(jnp.int32)
    cw_ref[...] = cwk.T


def _tc_call(x, W, ei3, edge_weight, np_):
    n, d_in = x.shape
    d_out = W.shape[1]
    deg = ei3.shape[2]
    bn = 256
    grid = np_ // bn
    return pl.pallas_call(
        functools.partial(_tc_body, n, deg),
        grid=(grid,),
        in_specs=[
            pl.BlockSpec((bn, d_in), lambda i: (i, 0)),
            pl.BlockSpec((d_in, d_out), lambda i: (0, 0)),
            pl.BlockSpec((2, bn, deg), lambda i: (0, i, 0)),
            pl.BlockSpec((bn, deg), lambda i: (i, 0)),
        ],
        out_specs=[
            pl.BlockSpec((bn, d_out), lambda i: (i, 0)),
            pl.BlockSpec((bn, K), lambda i: (i, 0)),
            pl.BlockSpec((bn, K), lambda i: (i, 0)),
        ],
        out_shape=[
            jax.ShapeDtypeStruct((n, d_out), jnp.float32),
            jax.ShapeDtypeStruct((np_, K), jnp.int32),
            jax.ShapeDtypeStruct((np_, K), jnp.float32),
        ],
    )(x, W, ei3, edge_weight)


def _splat(vec, k):
    idx = jnp.full((LANES, 1), k, dtype=jnp.int32)
    dn = lax.GatherDimensionNumbers(
        offset_dims=(), collapsed_slice_dims=(0,), start_index_map=(0,))
    return lax.gather(vec, idx, dn, slice_sizes=(1,),
                      mode=lax.GatherScatterMode.PROMISE_IN_BOUNDS)


def _sc_call(h, idx_flat, w_rep):
    n, d_out = h.shape
    np_ = w_rep.shape[0]
    info = plsc.get_sparse_core_info()
    nc, ns = info.num_cores, info.num_subcores
    nw = nc * ns
    pt = np_ // nw
    c = 16
    r = c * K
    t = pt // c
    nvec = d_out // LANES
    mesh = plsc.VectorSubcoreMesh(core_axis_name="c", subcore_axis_name="s")
    assert n % c == 0

    @functools.partial(
        pl.kernel,
        mesh=mesh,
        out_type=[
            jax.ShapeDtypeStruct((n, d_out), jnp.float32),
            jax.ShapeDtypeStruct((c, d_out), jnp.float32),
        ],
        scratch_types=[
            pltpu.VMEM((pt * K,), jnp.int32),
            pltpu.VMEM((2, r, d_out), jnp.float32),
            pltpu.VMEM((pt, K), jnp.float32),
            pltpu.VMEM((2, c, d_out), jnp.float32),
            pltpu.SemaphoreType.DMA((2,)),
            pltpu.SemaphoreType.DMA((2,)),
        ],
    )
    def sc_k(h_hbm, idx_hbm, w_hbm, out_hbm, trash_hbm, idx_v, rows_v, w_v,
             out_v, gsem, osem):
        wid = lax.axis_index("s") * nc + lax.axis_index("c")
        wnode0 = wid * pt

        def fetch(tt, b):
            pltpu.async_copy(
                h_hbm.at[idx_v.at[pl.ds(tt * r, r)]], rows_v.at[b], gsem.at[b])

        def put(tt, b):
            ow = wnode0 + tt * c

            @pl.when(ow < n)
            def _():
                pltpu.async_copy(out_v.at[b], out_hbm.at[pl.ds(ow, c)],
                                 osem.at[b])

            @pl.when(ow >= n)
            def _():
                pltpu.async_copy(out_v.at[b], trash_hbm, osem.at[b])

        pltpu.sync_copy(idx_hbm.at[pl.ds(wnode0 * K, pt * K)], idx_v)
        pltpu.sync_copy(w_hbm.at[pl.ds(wnode0, pt)], w_v)
        fetch(0, 0)

        def pair_body(t2, carry):
            for b in range(2):
                tt = t2 * 2 + b
                ob = 1 - b

                @pl.when(tt + 1 < t)
                def _():
                    fetch(tt + 1, ob)

                pltpu.make_async_copy(
                    h_hbm.at[idx_v.at[pl.ds(tt * r, r)]], rows_v.at[b],
                    gsem.at[b]).wait()

                def node_body(nn, carry2):
                    acc = [None] * nvec
                    wrow = w_v[tt * c + nn, :]
                    for kk in range(K):
                        row = nn * K + kk
                        wsplat = _splat(wrow, kk)
                        for cc in range(nvec):
                            term = wsplat * rows_v[b, row, pl.ds(cc * LANES, LANES)]
                            acc[cc] = term if kk == 0 else acc[cc] + term
                    for cc in range(nvec):
                        out_v[b, nn, pl.ds(cc * LANES, LANES)] = acc[cc]
                    return carry2

                lax.fori_loop(0, c, node_body, 0)

                @pl.when(tt >= 2)
                def _():
                    pltpu.make_async_copy(
                        out_v.at[b],
                        out_hbm.at[pl.ds(wnode0, c)],
                        osem.at[b]).wait()

                put(tt, b)
            return carry

        lax.fori_loop(0, t // 2, pair_body, 0)
        for b in range(2):
            pltpu.make_async_copy(
                out_v.at[b], out_hbm.at[pl.ds(wnode0, c)],
                osem.at[b]).wait()

    return sc_k(h, idx_flat, w_rep)[0]


def kernel(x, edge_index, edge_weight, W, a):
    n, d_in = x.shape
    e = edge_index.shape[1]
    deg = e // n
    nw = 32
    c = 8
    np_ = ((n + nw * c - 1) // (nw * c)) * (nw * c)

    h, cn, cw = _tc_call(x, W, edge_index.reshape(2, n, deg),
                         edge_weight.reshape(n, deg), np_)
    return _sc_call(h, cn.reshape(-1), cw)
